# Initial kernel scaffold; baseline (speedup 1.0000x reference)
import jax
import jax.numpy as jnp
from jax.experimental import pallas as pl


def kernel(y, left):
    raise NotImplementedError("write your pallas kernel here")


# speedup vs baseline: 6.1186x; 6.1186x over previous
import functools

import jax
import jax.numpy as jnp
from jax import lax
from jax.experimental import pallas as pl
from jax.experimental.pallas import tpu as pltpu
from jax.experimental.pallas import tpu_sc as plsc

B, S, C, T = 16, 4, 2, 131072
R = B * S * C
P = B * S
NW = 32
RPW = R // NW
CHB = 65536
NCH = T // CHB


def _flip_body(y_hbm, left_hbm, out_hbm, left_v, buf_v, sem):
    cid = lax.axis_index("c")
    sid = lax.axis_index("s")
    w = sid * 2 + cid
    base = w * RPW

    pltpu.sync_copy(left_hbm, left_v.at[pl.ds(0, P)])
    pair0 = base // 2
    lv = left_v[pl.ds(pair0, 16)]
    l0 = lv[0]
    l1 = lv[1]

    srcs = [base + l0, base + 1 - l0, base + 2 + l1, base + 3 - l1]
    for k in range(RPW):
        for j in range(NCH):
            col = j * CHB
            pltpu.async_copy(
                y_hbm.at[srcs[k], pl.ds(col, CHB)], buf_v, sem
            ).wait()
            pltpu.sync_copy(buf_v, out_hbm.at[base + k, pl.ds(col, CHB)])


@jax.jit
def _flip(y2, lf):
    mesh = plsc.VectorSubcoreMesh(core_axis_name="c", subcore_axis_name="s")
    return pl.kernel(
        _flip_body,
        out_type=jax.ShapeDtypeStruct((R, T), jnp.float32),
        mesh=mesh,
        scratch_types=[
            pltpu.VMEM((P + 16,), jnp.int32),
            pltpu.VMEM((CHB,), jnp.float32),
            pltpu.SemaphoreType.DMA,
        ],
    )(y2, lf)


def kernel(y, left):
    y2 = y.reshape(R, T)
    lf = left.reshape(P).astype(jnp.int32)
    out = _flip(y2, lf)
    return out.reshape(B, S, C, T)
